# baseline (speedup 1.0000x reference)
import jax
import jax.numpy as jnp
import numpy as np
from jax.experimental import pallas as pl
from jax.experimental.pallas import tpu as pltpu

_NR = 8
_NW = 8
_H = 48
_NB = _NR * _NW
_N = 2048
_FC_OUT = 128
_BI = 16
_BR = 256


def _pairwise_bins(ydash):
    r = jnp.linalg.norm(ydash[:, None, :] - ydash[None, :, :], axis=2)
    ring_ids = jnp.ceil((_NR - 1) * (jnp.log(r / 0.5) / 3.0))
    ring_ids = jnp.where(jnp.isneginf(ring_ids), 0.0, ring_ids)
    ring_ids = ring_ids.astype(jnp.int32)
    x_diff = ydash[:, 0] - ydash[:, 0][:, None]
    y_diff = ydash[:, 1] - ydash[:, 1][:, None]
    theta = jnp.arctan2(y_diff, x_diff)
    wedge_ids = theta * _NW / (2 * np.pi)
    wedge_ids = (wedge_ids + (_NW // 2 - 1)).astype(jnp.int32)
    fin = ring_ids * _NW + wedge_ids
    fin = jnp.where(fin < _NW, 0, fin)
    fin = jnp.where(fin >= _NW * _NW, 0, fin)
    return fin.astype(jnp.int32)


def _means_step(fin_ref, hid, m_ref):
    fin = fin_ref[...]

    k_iota = jax.lax.broadcasted_iota(jnp.int32, (_BI, _NB, _N), 1)
    onehot = (fin[:, None, :] == k_iota + _NW).astype(jnp.float32)
    onehot = onehot.reshape(_BI * _NB, _N)

    s = jnp.dot(onehot, hid[...], preferred_element_type=jnp.float32)
    cnt = s[:, _H:_H + 1]
    recip = 1.0 / jnp.maximum(cnt, 1.0)
    m_ref[...] = s[:, :_H] * recip


def _fc_step(m, Wt, b, out_ref):
    o = jnp.dot(m[...], Wt[...], preferred_element_type=jnp.float32) + b[...]
    out_ref[...] = jnp.maximum(o, 0.0)


def kernel(y_pred, x_start, hidden, W, b):
    del x_start
    fin = _pairwise_bins(jax.lax.stop_gradient(y_pred))

    hidden_aug = jnp.concatenate(
        [hidden, jnp.ones((_N, 1), hidden.dtype),
         jnp.zeros((_N, 64 - _H - 1), hidden.dtype)], axis=1)

    means = pl.pallas_call(
        _means_step,
        grid=(_N // _BI,),
        in_specs=[
            pl.BlockSpec((_BI, _N), lambda i: (i, 0)),
            pl.BlockSpec((_N, 64), lambda i: (0, 0)),
        ],
        out_specs=pl.BlockSpec((_BI * _NB, _H), lambda i: (i, 0)),
        out_shape=jax.ShapeDtypeStruct((_N * _NB, _H), jnp.float32),
    )(fin, hidden_aug)

    m2 = means.reshape(_N, _NB * _H)

    Wt = W.T
    b2 = b.reshape(1, _FC_OUT)
    return pl.pallas_call(
        _fc_step,
        grid=(_N // _BR,),
        in_specs=[
            pl.BlockSpec((_BR, _NB * _H), lambda i: (i, 0)),
            pl.BlockSpec((_NB * _H, _FC_OUT), lambda i: (0, 0)),
            pl.BlockSpec((1, _FC_OUT), lambda i: (0, 0)),
        ],
        out_specs=pl.BlockSpec((_BR, _FC_OUT), lambda i: (i, 0)),
        out_shape=jax.ShapeDtypeStruct((_N, _FC_OUT), jnp.float32),
    )(m2, Wt, b2)
